# Initial kernel scaffold; baseline (speedup 1.0000x reference)
import functools

import jax
import jax.numpy as jnp
from jax import lax
from jax.experimental import pallas as pl
from jax.experimental.pallas import tpu as pltpu
from jax.experimental.pallas import tpu_sc as plsc

_N = 10000
_E = 320000
_D = 128
_TILES_PER_SC = 16
_NUM_SC = 2
_EDGES_PER_TILE = _E // _TILES_PER_SC
_CHUNK = 80
_NCHUNK = _EDGES_PER_TILE // _CHUNK
_ROWS_PER_TILE = _N // _TILES_PER_SC


def _sc_aggregate_kernel(bases_hbm, src_hbm, dst_hbm, zeros_hbm, zeros8_hbm,
                         ones8_hbm, sums_hbm, cnts_hbm,
                         src_v, dst_v, rows_v, ones_v, acc_sh, cnt_sh, sem):
    c = lax.axis_index("c")
    s = lax.axis_index("s")
    wid = c * _TILES_PER_SC + s

    pltpu.sync_copy(zeros_hbm, acc_sh.at[pl.ds(s * _ROWS_PER_TILE, _ROWS_PER_TILE)])
    pltpu.sync_copy(zeros8_hbm, cnt_sh.at[pl.ds(s * _ROWS_PER_TILE, _ROWS_PER_TILE)])
    pltpu.sync_copy(src_hbm.at[pl.ds(wid * _NCHUNK, _NCHUNK)], src_v)
    pltpu.sync_copy(dst_hbm.at[pl.ds(wid * _NCHUNK, _NCHUNK)], dst_v)
    pltpu.sync_copy(ones8_hbm, ones_v)
    plsc.subcore_barrier()

    def body(j, carry):
        pltpu.async_copy(bases_hbm.at[src_v.at[j]], rows_v, sem).wait()
        pltpu.sync_copy(rows_v, acc_sh.at[dst_v.at[j]], add=True)
        pltpu.sync_copy(ones_v, cnt_sh.at[dst_v.at[j]], add=True)
        return carry

    lax.fori_loop(0, _NCHUNK, body, 0)
    plsc.subcore_barrier()

    out_base = c * _N + s * _ROWS_PER_TILE
    pltpu.sync_copy(acc_sh.at[pl.ds(s * _ROWS_PER_TILE, _ROWS_PER_TILE)],
                    sums_hbm.at[pl.ds(out_base, _ROWS_PER_TILE)])
    pltpu.sync_copy(cnt_sh.at[pl.ds(s * _ROWS_PER_TILE, _ROWS_PER_TILE)],
                    cnts_hbm.at[pl.ds(out_base, _ROWS_PER_TILE)])


_sc_aggregate = functools.partial(
    pl.kernel,
    out_type=(
        jax.ShapeDtypeStruct((_NUM_SC * _N, _D), jnp.float32),
        jax.ShapeDtypeStruct((_NUM_SC * _N, 8), jnp.float32),
    ),
    mesh=plsc.VectorSubcoreMesh(core_axis_name="c", subcore_axis_name="s"),
    scratch_types=[
        pltpu.VMEM((_NCHUNK, _CHUNK), jnp.int32),
        pltpu.VMEM((_NCHUNK, _CHUNK), jnp.int32),
        pltpu.VMEM((_CHUNK, _D), jnp.float32),
        pltpu.VMEM((_CHUNK, 8), jnp.float32),
        pltpu.VMEM_SHARED((_N, _D), jnp.float32),
        pltpu.VMEM_SHARED((_N, 8), jnp.float32),
        pltpu.SemaphoreType.DMA,
    ],
)(_sc_aggregate_kernel)


def _tc_dense_kernel(sp_ref, sn_ref, cp_ref, cn_ref, x_ref, w_ref, b_ref, o_ref):
    cp = jnp.maximum(cp_ref[...][:, 0:1], 1.0)
    cn = jnp.maximum(cn_ref[...][:, 0:1], 1.0)
    p = sp_ref[...] / cp
    q = sn_ref[...] / cn
    w = w_ref[...]
    hi = lax.Precision.HIGHEST
    h = (jnp.dot(p, w[0:_D], preferred_element_type=jnp.float32, precision=hi)
         + jnp.dot(q, w[_D:2 * _D], preferred_element_type=jnp.float32, precision=hi)
         + jnp.dot(x_ref[...], w[2 * _D:3 * _D], preferred_element_type=jnp.float32,
                   precision=hi)
         + b_ref[...])
    nrm = jnp.sqrt(jnp.sum(h * h, axis=-1, keepdims=True))
    o_ref[...] = h / jnp.maximum(nrm, 1e-12)


def kernel(x, base_pos, base_neg, edge_index_pos, edge_index_neg, W, b):
    n = x.shape[0]
    bases = jnp.concatenate([base_pos, base_neg], axis=0)
    src = jnp.concatenate([edge_index_pos[0], edge_index_neg[0] + n])
    dst = jnp.concatenate([edge_index_pos[1], edge_index_neg[1]])
    src_rs = src.reshape(-1, _CHUNK)
    dst_rs = dst.reshape(-1, _CHUNK)
    zeros = jnp.zeros((_ROWS_PER_TILE, _D), jnp.float32)
    zeros8 = jnp.zeros((_ROWS_PER_TILE, 8), jnp.float32)
    ones8 = jnp.ones((_CHUNK, 8), jnp.float32)

    sums, cnts = _sc_aggregate(bases, src_rs, dst_rs, zeros, zeros8, ones8)

    blk = 1000
    grid = (n // blk,)
    out = pl.pallas_call(
        _tc_dense_kernel,
        grid=grid,
        in_specs=[
            pl.BlockSpec((blk, _D), lambda i: (i, 0)),
            pl.BlockSpec((blk, _D), lambda i: (i, 0)),
            pl.BlockSpec((blk, 8), lambda i: (i, 0)),
            pl.BlockSpec((blk, 8), lambda i: (i, 0)),
            pl.BlockSpec((blk, _D), lambda i: (i, 0)),
            pl.BlockSpec((3 * _D, _D), lambda i: (0, 0)),
            pl.BlockSpec((1, _D), lambda i: (0, 0)),
        ],
        out_specs=pl.BlockSpec((blk, _D), lambda i: (i, 0)),
        out_shape=jax.ShapeDtypeStruct((n, _D), jnp.float32),
    )(sums[:n], sums[n:], cnts[:n], cnts[n:], x, W, b.reshape(1, _D))
    return out


# speedup vs baseline: 5.7591x; 5.7591x over previous
import functools

import jax
import jax.numpy as jnp
from jax import lax
from jax.experimental import pallas as pl
from jax.experimental.pallas import tpu as pltpu
from jax.experimental.pallas import tpu_sc as plsc

_N = 10000
_N_PAD = 10240
_E = 320000
_D = 128
_TILES_PER_SC = 16
_NUM_SC = 2
_CHUNK = 128
_BLK = 2
_NBLOCKS = _E // (_CHUNK * _BLK)
_ROWS_PER_TILE = _N_PAD // _TILES_PER_SC
_WB = 16


def _sc_aggregate_kernel(bpos_hbm, bneg_hbm, srcp_hbm, dstp_hbm, srcn_hbm,
                         dstn_hbm, sums_hbm,
                         src_v, dst_v, rows_v, acc_sh, sem):
    c = lax.axis_index("c")
    s = lax.axis_index("s")
    lanes = jnp.arange(16, dtype=jnp.int32)

    z = jnp.zeros((16,), jnp.float32)
    for r in range(_WB):
        for k in range(_D // 16):
            rows_v[r, pl.ds(k * 16, 16)] = z

    def zinit(t, carry):
        ids = s * _ROWS_PER_TILE + t * _WB + lanes
        pltpu.sync_copy(rows_v.at[pl.ds(0, _WB)], acc_sh.at[ids])
        return carry

    lax.fori_loop(0, _ROWS_PER_TILE // _WB, zinit, 0)
    plsc.subcore_barrier()

    def group(g, carry):
        blk = g * _TILES_PER_SC + s

        @pl.when(c == 0)
        def _():
            pltpu.sync_copy(srcp_hbm.at[blk], src_v)
            pltpu.sync_copy(dstp_hbm.at[blk], dst_v)

        @pl.when(c == 1)
        def _():
            pltpu.sync_copy(srcn_hbm.at[blk], src_v)
            pltpu.sync_copy(dstn_hbm.at[blk], dst_v)

        def body(j, carry2):
            @pl.when(c == 0)
            def _():
                pltpu.async_copy(bpos_hbm.at[src_v.at[j]], rows_v, sem).wait()

            @pl.when(c == 1)
            def _():
                pltpu.async_copy(bneg_hbm.at[src_v.at[j]], rows_v, sem).wait()

            pltpu.sync_copy(rows_v, acc_sh.at[dst_v.at[j]], add=True)
            return carry2

        lax.fori_loop(0, _BLK, body, 0)
        return carry

    nblk = jnp.where(s < _NBLOCKS % _TILES_PER_SC,
                     _NBLOCKS // _TILES_PER_SC + 1,
                     _NBLOCKS // _TILES_PER_SC)
    lax.fori_loop(0, nblk, group, 0)
    plsc.subcore_barrier()

    out_base = c * _N_PAD + s * _ROWS_PER_TILE

    def wb(t, carry):
        dst0 = out_base + t * _WB
        ids = s * _ROWS_PER_TILE + t * _WB + lanes
        pltpu.async_copy(acc_sh.at[ids], rows_v.at[pl.ds(0, _WB)], sem).wait()
        pltpu.sync_copy(rows_v.at[pl.ds(0, _WB)], sums_hbm.at[pl.ds(dst0, _WB)])
        return carry

    lax.fori_loop(0, _ROWS_PER_TILE // _WB, wb, 0)


_sc_aggregate = functools.partial(
    pl.kernel,
    out_type=jax.ShapeDtypeStruct((_NUM_SC * _N_PAD, _D), jnp.float32),
    mesh=plsc.VectorSubcoreMesh(core_axis_name="c", subcore_axis_name="s"),
    scratch_types=[
        pltpu.VMEM((_BLK, _CHUNK), jnp.int32),
        pltpu.VMEM((_BLK, _CHUNK), jnp.int32),
        pltpu.VMEM((_CHUNK, _D), jnp.float32),
        pltpu.VMEM_SHARED((_N_PAD, _D), jnp.float32),
        pltpu.SemaphoreType.DMA,
    ],
)(_sc_aggregate_kernel)


def _sc_count_kernel(dstp_hbm, dstn_hbm, cnts_hbm,
                     dst_v, ones_v, cacc_sh, sem):
    c = lax.axis_index("c")
    s = lax.axis_index("s")
    lanes = jnp.arange(16, dtype=jnp.int32)

    z = jnp.zeros((16,), jnp.float32)

    def zfill(r, carry):
        for k in range(_D // 16):
            ones_v[r, pl.ds(k * 16, 16)] = z
        return carry

    lax.fori_loop(0, _WB, zfill, 0)

    def zinit(t, carry):
        ids = s * _ROWS_PER_TILE + t * _WB + lanes
        pltpu.sync_copy(ones_v.at[pl.ds(0, _WB)], cacc_sh.at[ids])
        return carry

    lax.fori_loop(0, _ROWS_PER_TILE // _WB, zinit, 0)
    o = jnp.ones((16,), jnp.float32)

    def ofill(r, carry):
        for k in range(_D // 16):
            ones_v[r, pl.ds(k * 16, 16)] = o
        return carry

    lax.fori_loop(0, _CHUNK, ofill, 0)
    plsc.subcore_barrier()

    def group(g, carry):
        blk = g * _TILES_PER_SC + s

        @pl.when(c == 0)
        def _():
            pltpu.sync_copy(dstp_hbm.at[blk], dst_v)

        @pl.when(c == 1)
        def _():
            pltpu.sync_copy(dstn_hbm.at[blk], dst_v)

        def body(j, carry2):
            pltpu.sync_copy(ones_v, cacc_sh.at[dst_v.at[j]], add=True)
            return carry2

        lax.fori_loop(0, _BLK, body, 0)
        return carry

    nblk = jnp.where(s < _NBLOCKS % _TILES_PER_SC,
                     _NBLOCKS // _TILES_PER_SC + 1,
                     _NBLOCKS // _TILES_PER_SC)
    lax.fori_loop(0, nblk, group, 0)
    plsc.subcore_barrier()

    out_base = c * _N_PAD + s * _ROWS_PER_TILE

    def wb(t, carry):
        dst0 = out_base + t * _WB
        ids = s * _ROWS_PER_TILE + t * _WB + lanes
        pltpu.async_copy(cacc_sh.at[ids], ones_v.at[pl.ds(0, _WB)], sem).wait()
        pltpu.sync_copy(ones_v.at[pl.ds(0, _WB)], cnts_hbm.at[pl.ds(dst0, _WB)])
        return carry

    lax.fori_loop(0, _ROWS_PER_TILE // _WB, wb, 0)


_sc_count = functools.partial(
    pl.kernel,
    out_type=jax.ShapeDtypeStruct((_NUM_SC * _N_PAD, _D), jnp.float32),
    mesh=plsc.VectorSubcoreMesh(core_axis_name="c", subcore_axis_name="s"),
    scratch_types=[
        pltpu.VMEM((_BLK, _CHUNK), jnp.int32),
        pltpu.VMEM((_CHUNK, _D), jnp.float32),
        pltpu.VMEM_SHARED((_N_PAD, _D), jnp.float32),
        pltpu.SemaphoreType.DMA,
    ],
)(_sc_count_kernel)


def _tc_dense_kernel(sp_ref, sn_ref, cp_ref, cn_ref, x_ref, w_ref, b_ref, o_ref):
    cp = jnp.maximum(cp_ref[...][:, 0:1], 1.0)
    cn = jnp.maximum(cn_ref[...][:, 0:1], 1.0)
    p = sp_ref[...] / cp
    q = sn_ref[...] / cn
    w = w_ref[...]
    hi = lax.Precision.HIGHEST
    h = (jnp.dot(p, w[0:_D], preferred_element_type=jnp.float32, precision=hi)
         + jnp.dot(q, w[_D:2 * _D], preferred_element_type=jnp.float32, precision=hi)
         + jnp.dot(x_ref[...], w[2 * _D:3 * _D], preferred_element_type=jnp.float32,
                   precision=hi)
         + b_ref[...])
    nrm = jnp.sqrt(jnp.sum(h * h, axis=-1, keepdims=True))
    o_ref[...] = h / jnp.maximum(nrm, 1e-12)


def kernel(x, base_pos, base_neg, edge_index_pos, edge_index_neg, W, b):
    n = x.shape[0]
    srcp = edge_index_pos[0].reshape(_NBLOCKS, _BLK, _CHUNK)
    dstp = edge_index_pos[1].reshape(_NBLOCKS, _BLK, _CHUNK)
    srcn = edge_index_neg[0].reshape(_NBLOCKS, _BLK, _CHUNK)
    dstn = edge_index_neg[1].reshape(_NBLOCKS, _BLK, _CHUNK)
    sums = _sc_aggregate(base_pos, base_neg, srcp, dstp, srcn, dstn)
    cnts = _sc_count(dstp, dstn)

    blk = 1024
    grid = ((n + blk - 1) // blk,)
    out = pl.pallas_call(
        _tc_dense_kernel,
        grid=grid,
        in_specs=[
            pl.BlockSpec((blk, _D), lambda i: (i, 0)),
            pl.BlockSpec((blk, _D), lambda i: (i, 0)),
            pl.BlockSpec((blk, _D), lambda i: (i, 0)),
            pl.BlockSpec((blk, _D), lambda i: (i, 0)),
            pl.BlockSpec((blk, _D), lambda i: (i, 0)),
            pl.BlockSpec((3 * _D, _D), lambda i: (0, 0)),
            pl.BlockSpec((1, _D), lambda i: (0, 0)),
        ],
        out_specs=pl.BlockSpec((blk, _D), lambda i: (i, 0)),
        out_shape=jax.ShapeDtypeStruct((n, _D), jnp.float32),
    )(sums[:n], sums[_N_PAD:_N_PAD + n], cnts[:n], cnts[_N_PAD:_N_PAD + n],
      x, W, b.reshape(1, _D))
    return out


# speedup vs baseline: 6.2149x; 1.0791x over previous
import functools

import jax
import jax.numpy as jnp
from jax import lax
from jax.experimental import pallas as pl
from jax.experimental.pallas import tpu as pltpu
from jax.experimental.pallas import tpu_sc as plsc

_N = 10000
_N_PAD = 10240
_E = 320000
_D = 128
_TILES_PER_SC = 16
_NUM_SC = 2
_CHUNK = 128
_BLK = 4
_NBLOCKS = _E // (_CHUNK * _BLK)
_ROWS_PER_TILE = _N_PAD // _TILES_PER_SC
_WB = 16


def _sc_aggregate_kernel(bpos_hbm, bneg_hbm, srcp_hbm, dstp_hbm, srcn_hbm,
                         dstn_hbm, sums_hbm,
                         src_v, dst_v, rows_v, acc_sh, sem):
    c = lax.axis_index("c")
    s = lax.axis_index("s")
    lanes = jnp.arange(16, dtype=jnp.int32)

    z = jnp.zeros((16,), jnp.float32)
    for r in range(_WB):
        for k in range(_D // 16):
            rows_v[r, pl.ds(k * 16, 16)] = z

    def zinit(t, carry):
        ids = s * _ROWS_PER_TILE + t * _WB + lanes
        pltpu.sync_copy(rows_v.at[pl.ds(0, _WB)], acc_sh.at[ids])
        return carry

    lax.fori_loop(0, _ROWS_PER_TILE // _WB, zinit, 0)
    plsc.subcore_barrier()

    def group(g, carry):
        blk = g * _TILES_PER_SC + s

        @pl.when(c == 0)
        def _():
            pltpu.sync_copy(srcp_hbm.at[blk], src_v)
            pltpu.sync_copy(dstp_hbm.at[blk], dst_v)

        @pl.when(c == 1)
        def _():
            pltpu.sync_copy(srcn_hbm.at[blk], src_v)
            pltpu.sync_copy(dstn_hbm.at[blk], dst_v)

        def body(j, carry2):
            @pl.when(c == 0)
            def _():
                pltpu.async_copy(bpos_hbm.at[src_v.at[j]], rows_v, sem).wait()

            @pl.when(c == 1)
            def _():
                pltpu.async_copy(bneg_hbm.at[src_v.at[j]], rows_v, sem).wait()

            pltpu.sync_copy(rows_v, acc_sh.at[dst_v.at[j]], add=True)
            return carry2

        lax.fori_loop(0, _BLK, body, 0)
        return carry

    nblk = jnp.where(s < _NBLOCKS % _TILES_PER_SC,
                     _NBLOCKS // _TILES_PER_SC + 1,
                     _NBLOCKS // _TILES_PER_SC)
    lax.fori_loop(0, nblk, group, 0)
    plsc.subcore_barrier()

    out_base = c * _N_PAD + s * _ROWS_PER_TILE

    def wb(t, carry):
        dst0 = out_base + t * _WB
        ids = s * _ROWS_PER_TILE + t * _WB + lanes
        pltpu.async_copy(acc_sh.at[ids], rows_v.at[pl.ds(0, _WB)], sem).wait()
        pltpu.sync_copy(rows_v.at[pl.ds(0, _WB)], sums_hbm.at[pl.ds(dst0, _WB)])
        return carry

    lax.fori_loop(0, _ROWS_PER_TILE // _WB, wb, 0)


_sc_aggregate = functools.partial(
    pl.kernel,
    out_type=jax.ShapeDtypeStruct((_NUM_SC * _N_PAD, _D), jnp.float32),
    mesh=plsc.VectorSubcoreMesh(core_axis_name="c", subcore_axis_name="s"),
    scratch_types=[
        pltpu.VMEM((_BLK, _CHUNK), jnp.int32),
        pltpu.VMEM((_BLK, _CHUNK), jnp.int32),
        pltpu.VMEM((_CHUNK, _D), jnp.float32),
        pltpu.VMEM_SHARED((_N_PAD, _D), jnp.float32),
        pltpu.SemaphoreType.DMA,
    ],
)(_sc_aggregate_kernel)


def _sc_count_kernel(dstp_hbm, dstn_hbm, cnts_hbm,
                     dst_v, ones_v, cacc_sh, sem):
    c = lax.axis_index("c")
    s = lax.axis_index("s")
    lanes = jnp.arange(16, dtype=jnp.int32)

    z = jnp.zeros((16,), jnp.float32)

    def zfill(r, carry):
        for k in range(_D // 16):
            ones_v[r, pl.ds(k * 16, 16)] = z
        return carry

    lax.fori_loop(0, _WB, zfill, 0)

    def zinit(t, carry):
        ids = s * _ROWS_PER_TILE + t * _WB + lanes
        pltpu.sync_copy(ones_v.at[pl.ds(0, _WB)], cacc_sh.at[ids])
        return carry

    lax.fori_loop(0, _ROWS_PER_TILE // _WB, zinit, 0)
    o = jnp.ones((16,), jnp.float32)

    def ofill(r, carry):
        for k in range(_D // 16):
            ones_v[r, pl.ds(k * 16, 16)] = o
        return carry

    lax.fori_loop(0, _CHUNK, ofill, 0)
    plsc.subcore_barrier()

    def group(g, carry):
        blk = g * _TILES_PER_SC + s

        @pl.when(c == 0)
        def _():
            pltpu.sync_copy(dstp_hbm.at[blk], dst_v)

        @pl.when(c == 1)
        def _():
            pltpu.sync_copy(dstn_hbm.at[blk], dst_v)

        def body(j, carry2):
            pltpu.sync_copy(ones_v, cacc_sh.at[dst_v.at[j]], add=True)
            return carry2

        lax.fori_loop(0, _BLK, body, 0)
        return carry

    nblk = jnp.where(s < _NBLOCKS % _TILES_PER_SC,
                     _NBLOCKS // _TILES_PER_SC + 1,
                     _NBLOCKS // _TILES_PER_SC)
    lax.fori_loop(0, nblk, group, 0)
    plsc.subcore_barrier()

    out_base = c * _N_PAD + s * _ROWS_PER_TILE

    def wb(t, carry):
        dst0 = out_base + t * _WB
        ids = s * _ROWS_PER_TILE + t * _WB + lanes
        pltpu.async_copy(cacc_sh.at[ids], ones_v.at[pl.ds(0, _WB)], sem).wait()
        pltpu.sync_copy(ones_v.at[pl.ds(0, _WB)], cnts_hbm.at[pl.ds(dst0, _WB)])
        return carry

    lax.fori_loop(0, _ROWS_PER_TILE // _WB, wb, 0)


_sc_count = functools.partial(
    pl.kernel,
    out_type=jax.ShapeDtypeStruct((_NUM_SC * _N_PAD, _D), jnp.float32),
    mesh=plsc.VectorSubcoreMesh(core_axis_name="c", subcore_axis_name="s"),
    scratch_types=[
        pltpu.VMEM((_BLK, _CHUNK), jnp.int32),
        pltpu.VMEM((_CHUNK, _D), jnp.float32),
        pltpu.VMEM_SHARED((_N_PAD, _D), jnp.float32),
        pltpu.SemaphoreType.DMA,
    ],
)(_sc_count_kernel)


def _tc_dense_kernel(sp_ref, sn_ref, cp_ref, cn_ref, x_ref, w_ref, b_ref, o_ref):
    cp = jnp.maximum(cp_ref[...][:, 0:1], 1.0)
    cn = jnp.maximum(cn_ref[...][:, 0:1], 1.0)
    p = sp_ref[...] / cp
    q = sn_ref[...] / cn
    w = w_ref[...]
    hi = lax.Precision.HIGHEST
    h = (jnp.dot(p, w[0:_D], preferred_element_type=jnp.float32, precision=hi)
         + jnp.dot(q, w[_D:2 * _D], preferred_element_type=jnp.float32, precision=hi)
         + jnp.dot(x_ref[...], w[2 * _D:3 * _D], preferred_element_type=jnp.float32,
                   precision=hi)
         + b_ref[...])
    nrm = jnp.sqrt(jnp.sum(h * h, axis=-1, keepdims=True))
    o_ref[...] = h / jnp.maximum(nrm, 1e-12)


def kernel(x, base_pos, base_neg, edge_index_pos, edge_index_neg, W, b):
    n = x.shape[0]
    srcp = edge_index_pos[0].reshape(_NBLOCKS, _BLK, _CHUNK)
    dstp = edge_index_pos[1].reshape(_NBLOCKS, _BLK, _CHUNK)
    srcn = edge_index_neg[0].reshape(_NBLOCKS, _BLK, _CHUNK)
    dstn = edge_index_neg[1].reshape(_NBLOCKS, _BLK, _CHUNK)
    sums = _sc_aggregate(base_pos, base_neg, srcp, dstp, srcn, dstn)
    cnts = _sc_count(dstp, dstn)

    blk = 1024
    grid = ((n + blk - 1) // blk,)
    out = pl.pallas_call(
        _tc_dense_kernel,
        grid=grid,
        in_specs=[
            pl.BlockSpec((blk, _D), lambda i: (i, 0)),
            pl.BlockSpec((blk, _D), lambda i: (i, 0)),
            pl.BlockSpec((blk, _D), lambda i: (i, 0)),
            pl.BlockSpec((blk, _D), lambda i: (i, 0)),
            pl.BlockSpec((blk, _D), lambda i: (i, 0)),
            pl.BlockSpec((3 * _D, _D), lambda i: (0, 0)),
            pl.BlockSpec((1, _D), lambda i: (0, 0)),
        ],
        out_specs=pl.BlockSpec((blk, _D), lambda i: (i, 0)),
        out_shape=jax.ShapeDtypeStruct((n, _D), jnp.float32),
    )(sums[:n], sums[_N_PAD:_N_PAD + n], cnts[:n], cnts[_N_PAD:_N_PAD + n],
      x, W, b.reshape(1, _D))
    return out
